# Initial kernel scaffold; baseline (speedup 1.0000x reference)
import functools

import jax
import jax.numpy as jnp
from jax import lax
from jax.experimental import pallas as pl
from jax.experimental.pallas import tpu as pltpu
from jax.experimental.pallas import tpu_sc as plsc

SIZE = 64
BATCH = 4
PART = 18
NPAIR = BATCH * PART
GSIZE = 13
PAD = 6
Z_RES = (1, 2, 4, 64)
NC, NS = 2, 16
NW = NC * NS


def _sc_body(coords_hbm, g2_hbm, g3_hbm, o0, o1, o2, o3,
             coordsv, g2v, g3v, zbuf, pbuf2, pbuf3, zsem, psem):
    wid = lax.axis_index("c") * NS + lax.axis_index("s")
    lane = lax.iota(jnp.int32, 16)

    pltpu.sync_copy(coords_hbm, coordsv)
    pltpu.sync_copy(g2_hbm, g2v)
    pltpu.sync_copy(g3_hbm, g3v)

    zero16 = jnp.zeros((16,), jnp.float32)

    def _zb(i, carry):
        a = i // 256
        rem = i - a * 256
        b = rem // 4
        c = rem - b * 4
        zbuf[a, b, pl.ds(c * 16, 16)] = zero16
        return carry

    lax.fori_loop(0, 8 * 64 * 4, _zb, 0)

    def do_pair(pair):
        x0 = coordsv[pair, 0]
        y0 = coordsv[pair, 1]
        zz = coordsv[pair, 2]

        zh = []
        for k in range(8):
            zh.append(pltpu.async_copy(
                zbuf, o3.at[pl.ds(pair * 64 + k * 8, 8)], zsem))
        zh.append(pltpu.async_copy(
            zbuf.at[pl.ds(0, 4)], o2.at[pl.ds(pair * 4, 4)], zsem))
        zh.append(pltpu.async_copy(
            zbuf.at[pl.ds(0, 2)], o1.at[pl.ds(pair * 2, 2)], zsem))
        zh.append(pltpu.async_copy(
            zbuf.at[pl.ds(0, 1)], o0.at[pl.ds(pair, 1)], zsem))

        sx = 63 - x0
        yc = jnp.clip(y0 - PAD, 0, SIZE - GSIZE)
        syo = yc - y0 + PAD
        zidx3 = zz - 1
        zc = jnp.clip(zidx3 - PAD, 0, SIZE - GSIZE)
        szo = zc - zidx3 + PAD

        def body_jz(jz, carry):
            zs = szo + jz
            vz = (zs >= 0) & (zs < GSIZE)
            zcl = jnp.full((16,), jnp.clip(zs, 0, GSIZE - 1), jnp.int32)

            def body_j(j, c2):
                ys = syo + j
                vy = (ys >= 0) & (ys < GSIZE)
                ycl = jnp.full((16,), jnp.clip(ys, 0, GSIZE - 1), jnp.int32)
                scale = jnp.where(vz & vy, 1.0, 0.0).astype(jnp.float32)
                for i in range(4):
                    xi = sx + i * 16 + lane
                    v = plsc.load_gather(g3v, [zcl, ycl, xi]) * scale
                    pbuf3[jz, j, pl.ds(i * 16, 16)] = v
                return c2

            lax.fori_loop(0, GSIZE, body_j, 0)
            return carry

        lax.fori_loop(0, GSIZE, body_jz, 0)

        def body_j2(j, carry):
            ys = syo + j
            vy = (ys >= 0) & (ys < GSIZE)
            ycl = jnp.full((16,), jnp.clip(ys, 0, GSIZE - 1), jnp.int32)
            scale = jnp.where(vy, 1.0, 0.0).astype(jnp.float32)
            for r in range(3):
                rcl = jnp.full((16,), r, jnp.int32)
                for i in range(4):
                    xi = sx + i * 16 + lane
                    v = plsc.load_gather(g2v, [rcl, ycl, xi]) * scale
                    pbuf2[r, j, pl.ds(i * 16, 16)] = v
            return carry

        lax.fori_loop(0, GSIZE, body_j2, 0)

        for h in zh:
            h.wait()

        ph = pltpu.async_copy(
            pbuf3,
            o3.at[pl.ds(pair * 64 + zc, GSIZE), pl.ds(yc, GSIZE), :],
            psem)
        for r, (zr, oref) in enumerate(zip(Z_RES[:3], (o0, o1, o2))):
            zidx_r = (zz * zr + 63) // 64 - 1

            @pl.when(zidx_r >= 0)
            def _(r=r, zr=zr, oref=oref, zidx_r=zidx_r):
                pltpu.async_copy(
                    pbuf2.at[r],
                    oref.at[pair * zr + zidx_r, pl.ds(yc, GSIZE), :],
                    psem).wait()

        ph.wait()

    do_pair(wid)
    do_pair(wid + 32)

    @pl.when(wid + 64 < NPAIR)
    def _():
        do_pair(wid + 64)


@jax.jit
def kernel(coords, g0, g1, g2, g3):
    f32 = jnp.float32
    coords16 = jnp.zeros((NPAIR, 16), jnp.int32)
    coords16 = coords16.at[:, :3].set(coords.reshape(NPAIR, 3))
    g2x = jnp.zeros((3, GSIZE, 128), f32)
    g2x = g2x.at[:, :, 57:57 + GSIZE].set(
        jnp.stack([g0[0], g1[0], g2[0]]).astype(f32))
    g3x = jnp.zeros((GSIZE, GSIZE, 128), f32)
    g3x = g3x.at[:, :, 57:57 + GSIZE].set(g3.astype(f32))

    mesh = plsc.VectorSubcoreMesh(
        core_axis_name="c", subcore_axis_name="s",
        num_cores=NC, num_subcores=NS)
    out_type = [
        jax.ShapeDtypeStruct((NPAIR * 1, SIZE, SIZE), f32),
        jax.ShapeDtypeStruct((NPAIR * 2, SIZE, SIZE), f32),
        jax.ShapeDtypeStruct((NPAIR * 4, SIZE, SIZE), f32),
        jax.ShapeDtypeStruct((NPAIR * 64, SIZE, SIZE), f32),
    ]
    scratch = [
        pltpu.VMEM((NPAIR, 16), jnp.int32),
        pltpu.VMEM((3, GSIZE, 128), f32),
        pltpu.VMEM((GSIZE, GSIZE, 128), f32),
        pltpu.VMEM((8, SIZE, SIZE), f32),
        pltpu.VMEM((3, GSIZE, SIZE), f32),
        pltpu.VMEM((GSIZE, GSIZE, SIZE), f32),
        pltpu.SemaphoreType.DMA,
        pltpu.SemaphoreType.DMA,
    ]
    o0, o1, o2, o3 = pl.kernel(
        _sc_body, out_type=out_type, mesh=mesh, scratch_types=scratch,
    )(coords16, g2x, g3x)
    return (
        o0.reshape(BATCH, PART, 1, SIZE, SIZE),
        o1.reshape(BATCH, PART, 2, SIZE, SIZE),
        o2.reshape(BATCH, PART, 4, SIZE, SIZE),
        o3.reshape(BATCH, PART, 64, SIZE, SIZE),
    )


# speedup vs baseline: 1184.4619x; 1184.4619x over previous
import jax
import jax.numpy as jnp
from jax import lax
from jax.experimental import pallas as pl
from jax.experimental.pallas import tpu as pltpu
from jax.experimental.pallas import tpu_sc as plsc

SIZE = 64
BATCH = 4
PART = 18
NPAIR = BATCH * PART
GSIZE = 13
PAD = 6
Z_RES = (1, 2, 4, 64)
NC, NS = 2, 16
PLANE = SIZE * SIZE


def _zero_range(ref, nvec):
    zero16 = jnp.zeros((16,), jnp.float32)

    def body(i, carry):
        ref[pl.ds(i * 16, 16)] = zero16
        return carry

    lax.fori_loop(0, nvec, body, 0)


def _sc_body(coords_hbm, g2_hbm, g3_hbm, o0, o1, o2, o3,
             coordsv, g2v, g3v, zbuf, pbig2, pbig3, zsem, psem):
    wid = lax.axis_index("c") * NS + lax.axis_index("s")
    lane = lax.iota(jnp.int32, 16)
    zero16 = jnp.zeros((16,), jnp.float32)

    pltpu.sync_copy(coords_hbm, coordsv)
    pltpu.sync_copy(g2_hbm, g2v)
    pltpu.sync_copy(g3_hbm, g3v)

    _zero_range(zbuf, 4 * PLANE // 16)
    _zero_range(pbig2, 3 * PLANE // 16)
    _zero_range(pbig3, GSIZE * PLANE // 16)

    def do_pair(pair):
        crow = coordsv[pl.ds(pair * 16, 16)]
        x0 = crow[0]
        y0 = crow[1]
        zz = crow[2]

        zh = []
        for k in range(16):
            zh.append(pltpu.async_copy(
                zbuf, o3.at[pl.ds(pair * 64 * PLANE + k * 4 * PLANE,
                                  4 * PLANE)], zsem))
        zh.append(pltpu.async_copy(
            zbuf, o2.at[pl.ds(pair * 4 * PLANE, 4 * PLANE)], zsem))
        zh.append(pltpu.async_copy(
            zbuf.at[pl.ds(0, 2 * PLANE)],
            o1.at[pl.ds(pair * 2 * PLANE, 2 * PLANE)], zsem))
        zh.append(pltpu.async_copy(
            zbuf.at[pl.ds(0, PLANE)],
            o0.at[pl.ds(pair * PLANE, PLANE)], zsem))

        sx = 63 - x0
        yc = jnp.clip(y0 - PAD, 0, SIZE - GSIZE)
        syo = yc - y0 + PAD
        zidx3 = zz - 1
        zc = jnp.clip(zidx3 - PAD, 0, SIZE - GSIZE)
        szo = zc - zidx3 + PAD

        def body_jz(jz, carry):
            zs = szo + jz
            vz = (zs >= 0) & (zs < GSIZE)
            zcl = jnp.full((16,), jnp.clip(zs, 0, GSIZE - 1), jnp.int32)

            def body_j(j, c2):
                ys = syo + j
                vy = (ys >= 0) & (ys < GSIZE)
                ycl = jnp.full((16,), jnp.clip(ys, 0, GSIZE - 1), jnp.int32)
                scale = jnp.where(vz & vy, 1.0, 0.0).astype(jnp.float32)
                base = (jz * SIZE + yc + j) * SIZE
                for i in range(4):
                    xi = sx + i * 16 + lane
                    v = plsc.load_gather(
                        g3v, [(zcl * GSIZE + ycl) * 128 + xi]) * scale
                    pbig3[pl.ds(base + i * 16, 16)] = v
                return c2

            lax.fori_loop(0, GSIZE, body_j, 0)
            return carry

        lax.fori_loop(0, GSIZE, body_jz, 0)

        def body_j2(j, carry):
            ys = syo + j
            vy = (ys >= 0) & (ys < GSIZE)
            ycl = jnp.full((16,), jnp.clip(ys, 0, GSIZE - 1), jnp.int32)
            scale = jnp.where(vy, 1.0, 0.0).astype(jnp.float32)
            for r in range(3):
                rcl = jnp.full((16,), r, jnp.int32)
                base = (r * SIZE + yc + j) * SIZE
                for i in range(4):
                    xi = sx + i * 16 + lane
                    v = plsc.load_gather(
                        g2v, [(rcl * GSIZE + ycl) * 128 + xi]) * scale
                    pbig2[pl.ds(base + i * 16, 16)] = v
            return carry

        lax.fori_loop(0, GSIZE, body_j2, 0)

        for h in zh:
            h.wait()

        ph = [pltpu.async_copy(
            pbig3,
            o3.at[pl.ds((pair * 64 + zc) * PLANE, GSIZE * PLANE)],
            psem)]
        for r, (zr, oref) in enumerate(zip(Z_RES[:3], (o0, o1, o2))):
            zidx_r = (zz * zr + 63) // 64 - 1

            @pl.when(zidx_r >= 0)
            def _(r=r, zr=zr, oref=oref, zidx_r=zidx_r):
                pltpu.async_copy(
                    pbig2.at[pl.ds(r * PLANE, PLANE)],
                    oref.at[pl.ds((pair * zr + zidx_r) * PLANE, PLANE)],
                    psem).wait()

        for h in ph:
            h.wait()

        def rz3(jz, carry):
            def rzj(j, c2):
                base = (jz * SIZE + yc + j) * SIZE
                for i in range(4):
                    pbig3[pl.ds(base + i * 16, 16)] = zero16
                return c2
            lax.fori_loop(0, GSIZE, rzj, 0)
            return carry

        lax.fori_loop(0, GSIZE, rz3, 0)

        def rz2(j, carry):
            for r in range(3):
                base = (r * SIZE + yc + j) * SIZE
                for i in range(4):
                    pbig2[pl.ds(base + i * 16, 16)] = zero16
            return carry

        lax.fori_loop(0, GSIZE, rz2, 0)

    do_pair(wid)
    do_pair(wid + 32)

    @pl.when(wid + 64 < NPAIR)
    def _():
        do_pair(wid + 64)


@jax.jit
def kernel(coords, g0, g1, g2, g3):
    f32 = jnp.float32
    coords16 = jnp.zeros((NPAIR, 16), jnp.int32)
    coords16 = coords16.at[:, :3].set(coords.reshape(NPAIR, 3))
    g2x = jnp.zeros((3, GSIZE, 128), f32)
    g2x = g2x.at[:, :, 57:57 + GSIZE].set(
        jnp.stack([g0[0], g1[0], g2[0]]).astype(f32))
    g3x = jnp.zeros((GSIZE, GSIZE, 128), f32)
    g3x = g3x.at[:, :, 57:57 + GSIZE].set(g3.astype(f32))

    mesh = plsc.VectorSubcoreMesh(
        core_axis_name="c", subcore_axis_name="s",
        num_cores=NC, num_subcores=NS)
    out_type = [
        jax.ShapeDtypeStruct((NPAIR * 1 * PLANE,), f32),
        jax.ShapeDtypeStruct((NPAIR * 2 * PLANE,), f32),
        jax.ShapeDtypeStruct((NPAIR * 4 * PLANE,), f32),
        jax.ShapeDtypeStruct((NPAIR * 64 * PLANE,), f32),
    ]
    scratch = [
        pltpu.VMEM((NPAIR * 16,), jnp.int32),
        pltpu.VMEM((3 * GSIZE * 128,), f32),
        pltpu.VMEM((GSIZE * GSIZE * 128,), f32),
        pltpu.VMEM((4 * PLANE,), f32),
        pltpu.VMEM((3 * PLANE,), f32),
        pltpu.VMEM((GSIZE * PLANE,), f32),
        pltpu.SemaphoreType.DMA,
        pltpu.SemaphoreType.DMA,
    ]
    o0, o1, o2, o3 = pl.kernel(
        _sc_body, out_type=out_type, mesh=mesh, scratch_types=scratch,
        compiler_params=pltpu.CompilerParams(needs_layout_passes=False),
    )(coords16.reshape(-1), g2x.reshape(-1), g3x.reshape(-1))
    return (
        o0.reshape(BATCH, PART, 1, SIZE, SIZE),
        o1.reshape(BATCH, PART, 2, SIZE, SIZE),
        o2.reshape(BATCH, PART, 4, SIZE, SIZE),
        o3.reshape(BATCH, PART, 64, SIZE, SIZE),
    )
